# baseline (speedup 1.0000x reference)
import jax
import jax.numpy as jnp
from jax import lax
from jax.experimental import pallas as pl
from jax.experimental.pallas import tpu as pltpu, tpu_sc as plsc

D = 768
NC, NS, L = 2, 16, 16
NW = NC * NS
CH = 32


def _emb_body(batch, seq_len, idx_hbm, table_hbm, pe_hbm, out_hbm,
              idx_v, rows_v, pe_v, gsems, ssems):
    s_per_w = seq_len // NW
    nh = s_per_w // CH
    nchunk = nh * batch
    wid = lax.axis_index("s") * NC + lax.axis_index("c")

    pltpu.sync_copy(idx_hbm.at[wid], idx_v)

    def fetch(c, slot):
        return pltpu.make_async_copy(table_hbm.at[idx_v.at[c]],
                                     rows_v.at[slot], gsems[slot])

    def store(c, bb, h, slot):
        return pltpu.make_async_copy(
            rows_v.at[slot],
            out_hbm.at[pl.ds(wid * s_per_w + (bb * seq_len + h * CH), CH)],
            ssems[slot])

    for c in range(min(batch - 1, nchunk)):
        fetch(c, c % batch).start()

    def group(h, carry):
        pltpu.sync_copy(pe_hbm.at[pl.ds(wid * s_per_w + h * CH, CH)], pe_v)
        for bb in range(batch):
            c = h * batch + bb
            fetch(c, bb).wait()

            @plsc.parallel_loop(0, CH, unroll=4)
            def add_row(r):
                for j in range(D // L):
                    sl = pl.ds(j * L, L)
                    plsc.addupdate(rows_v.at[bb, r, sl], pe_v[r, sl])

            store(c, bb, h, bb).start()
            nslot = (bb + batch - 1) % batch

            @pl.when((c >= 1) & (c + batch - 1 < nchunk))
            def _drain():
                store(c - 1, 0, 0, nslot).wait()

            @pl.when(c + batch - 1 < nchunk)
            def _next():
                fetch(c + batch - 1, nslot).start()
        return carry

    lax.fori_loop(0, nh, group, 0)
    for bb in range(batch):
        store(0, 0, 0, bb).wait()


def kernel(x, token_table, pe):
    B, S = x.shape
    s_per_w = S // NW
    nh = s_per_w // CH
    xt = (x.reshape(B, NW, nh, CH).transpose(1, 2, 0, 3)
           .reshape(NW, nh * B, CH).astype(jnp.int32))
    pe_s = pe[:S]
    mesh = plsc.VectorSubcoreMesh(core_axis_name="c", subcore_axis_name="s",
                                  num_cores=NC, num_subcores=NS)

    def body(*refs):
        _emb_body(B, S, *refs)

    out = pl.kernel(
        body,
        out_type=jax.ShapeDtypeStruct((B * S, D), jnp.float32),
        mesh=mesh,
        scratch_types=[
            pltpu.VMEM((nh * B, CH), jnp.int32),
            pltpu.VMEM((B, CH, D), jnp.float32),
            pltpu.VMEM((CH, D), jnp.float32),
            [pltpu.SemaphoreType.DMA] * B,
            [pltpu.SemaphoreType.DMA] * B,
        ],
    )(xt, token_table, pe_s)
    return out.reshape(B, S, D)


# speedup vs baseline: 1.0001x; 1.0001x over previous
import jax
import jax.numpy as jnp
from jax import lax
from jax.experimental import pallas as pl
from jax.experimental.pallas import tpu as pltpu, tpu_sc as plsc

D = 768
NC, NS, L = 2, 16, 16
NW = NC * NS
CH = 16


def _emb_body(batch, seq_len, idx_hbm, table_hbm, pe_hbm, out_hbm,
              idx_v, rows_v, pe_v, gsems, ssems):
    s_per_w = seq_len // NW
    nh = s_per_w // CH
    wid = lax.axis_index("s") * NC + lax.axis_index("c")

    pltpu.sync_copy(idx_hbm.at[wid], idx_v)

    def fetch(g, bb, slot):
        return pltpu.make_async_copy(table_hbm.at[idx_v.at[g * batch + bb]],
                                     rows_v.at[slot], gsems[slot])

    def store(g, bb, slot):
        return pltpu.make_async_copy(
            rows_v.at[slot],
            out_hbm.at[pl.ds(wid * s_per_w + (bb * seq_len + g * CH), CH)],
            ssems[slot])

    for bb in range(batch):
        fetch(0, bb, bb).start()

    def pair(gp, carry):
        for p in range(2):
            g = gp * 2 + p
            base = p * batch
            pltpu.sync_copy(pe_hbm.at[pl.ds(wid * s_per_w + g * CH, CH)],
                            pe_v.at[p])
            for bb in range(batch):
                fetch(g, bb, base + bb).wait()
            obase = (1 - p) * batch

            @pl.when(g >= 1)
            def _drain():
                for bb in range(batch):
                    store(0, 0, obase + bb).wait()

            @pl.when(g + 1 < nh)
            def _next():
                for bb in range(batch):
                    fetch(g + 1, bb, obase + bb).start()

            @plsc.parallel_loop(0, CH, unroll=2)
            def add_row(r):
                for j in range(D // L):
                    sl = pl.ds(j * L, L)
                    v = pe_v[p, r, sl]
                    for bb in range(batch):
                        plsc.addupdate(rows_v.at[base + bb, r, sl], v)

            for bb in range(batch):
                store(g, bb, base + bb).start()
        return carry

    lax.fori_loop(0, nh // 2, pair, 0)
    for bb in range(batch):
        store(0, 0, ((nh - 1) % 2) * batch + bb).wait()


def kernel(x, token_table, pe):
    B, S = x.shape
    s_per_w = S // NW
    nh = s_per_w // CH
    xt = (x.reshape(B, NW, nh, CH).transpose(1, 2, 0, 3)
           .reshape(NW, nh * B, CH).astype(jnp.int32))
    pe_s = pe[:S]
    mesh = plsc.VectorSubcoreMesh(core_axis_name="c", subcore_axis_name="s",
                                  num_cores=NC, num_subcores=NS)

    def body(*refs):
        _emb_body(B, S, *refs)

    out = pl.kernel(
        body,
        out_type=jax.ShapeDtypeStruct((B * S, D), jnp.float32),
        mesh=mesh,
        scratch_types=[
            pltpu.VMEM((nh * B, CH), jnp.int32),
            pltpu.VMEM((2 * B, CH, D), jnp.float32),
            pltpu.VMEM((2, CH, D), jnp.float32),
            [pltpu.SemaphoreType.DMA] * (2 * B),
            [pltpu.SemaphoreType.DMA] * (2 * B),
        ],
    )(xt, token_table, pe_s)
    return out.reshape(B, S, D)


# speedup vs baseline: 1.1379x; 1.1378x over previous
import jax
import jax.numpy as jnp
from jax import lax
from jax.experimental import pallas as pl
from jax.experimental.pallas import tpu as pltpu, tpu_sc as plsc

D = 768
NC, NS, L = 2, 16, 16
NW = NC * NS
CH = 32


def _emb_body(batch, seq_len, idx_hbm, table_hbm, pe_hbm, out_hbm,
              idx_v, rows_v, pe_v, gsems, ssems):
    s_per_w = seq_len // NW
    nh = s_per_w // CH
    nchunk = nh * batch
    wid = lax.axis_index("s") * NC + lax.axis_index("c")

    pltpu.sync_copy(idx_hbm.at[wid], idx_v)

    def fetch(c, slot):
        return pltpu.make_async_copy(table_hbm.at[idx_v.at[c]],
                                     rows_v.at[slot], gsems[slot])

    def store(c, bb, h, slot):
        return pltpu.make_async_copy(
            rows_v.at[slot],
            out_hbm.at[pl.ds(wid * s_per_w + (bb * seq_len + h * CH), CH)],
            ssems[slot])

    for c in range(min(batch - 1, nchunk)):
        fetch(c, c % batch).start()

    def group(h, carry):
        pltpu.sync_copy(pe_hbm.at[pl.ds(wid * s_per_w + h * CH, CH)], pe_v)
        for bb in range(batch):
            c = h * batch + bb
            fetch(c, bb).wait()

            @plsc.parallel_loop(0, CH, unroll=2)
            def add_row(r):
                for j in range(D // L):
                    sl = pl.ds(j * L, L)
                    plsc.addupdate(rows_v.at[bb, r, sl], pe_v[r, sl])

            store(c, bb, h, bb).start()
            nslot = (bb + batch - 1) % batch

            @pl.when((c >= 1) & (c + batch - 1 < nchunk))
            def _drain():
                store(c - 1, 0, 0, nslot).wait()

            @pl.when(c + batch - 1 < nchunk)
            def _next():
                fetch(c + batch - 1, nslot).start()
        return carry

    lax.fori_loop(0, nh, group, 0)
    for bb in range(batch):
        store(0, 0, 0, bb).wait()


def kernel(x, token_table, pe):
    B, S = x.shape
    s_per_w = S // NW
    nh = s_per_w // CH
    xt = (x.reshape(B, NW, nh, CH).transpose(1, 2, 0, 3)
           .reshape(NW, nh * B, CH).astype(jnp.int32))
    pe_s = pe[:S]
    mesh = plsc.VectorSubcoreMesh(core_axis_name="c", subcore_axis_name="s",
                                  num_cores=NC, num_subcores=NS)

    def body(*refs):
        _emb_body(B, S, *refs)

    out = pl.kernel(
        body,
        out_type=jax.ShapeDtypeStruct((B * S, D), jnp.float32),
        mesh=mesh,
        scratch_types=[
            pltpu.VMEM((nh * B, CH), jnp.int32),
            pltpu.VMEM((B, CH, D), jnp.float32),
            pltpu.VMEM((CH, D), jnp.float32),
            [pltpu.SemaphoreType.DMA] * B,
            [pltpu.SemaphoreType.DMA] * B,
        ],
    )(xt, token_table, pe_s)
    return out.reshape(B, S, D)
